# baseline (speedup 1.0000x reference)
import functools

import jax
import jax.numpy as jnp
from jax import lax
from jax.experimental import pallas as pl
from jax.experimental.pallas import tpu as pltpu
from jax.experimental.pallas import tpu_sc as plsc

_B = 64
_S = 512
_DIM = 768
_NV = _DIM // 16

_NC = 2
_NS = 16
_NW = _NC * _NS
_SW = _S // _NW
_CB = 4
_NCHUNK = _B // _CB


def _lane_sum(v):
  for sh in (8, 4, 2, 1):
    perm = jnp.bitwise_xor(lax.iota(jnp.int32, 16), sh)
    v = v + v.at[perm].get(mode="promise_in_bounds", unique_indices=True)
  return v


def _ln_chunk(rows_v, buf, ptt_v, gb_v):
  @plsc.parallel_loop(0, _CB * _SW, unroll=2)
  def tok(t):
    s = jnp.bitwise_and(t, _SW - 1)
    acc = []
    sq = []
    for r in range(_NV):
      x = rows_v[buf, t, pl.ds(16 * r, 16)] + ptt_v[s, pl.ds(16 * r, 16)]
      rows_v[buf, t, pl.ds(16 * r, 16)] = x
      if r < 4:
        acc.append(x)
        sq.append(x * x)
      else:
        acc[r % 4] = acc[r % 4] + x
        sq[r % 4] = sq[r % 4] + x * x
    mv = _lane_sum((acc[0] + acc[1]) + (acc[2] + acc[3])) * (1.0 / _DIM)
    vv = (_lane_sum((sq[0] + sq[1]) + (sq[2] + sq[3])) * (1.0 / _DIM)
          - mv * mv + 1e-12)
    iv = lax.bitcast_convert_type(vv, jnp.int32)
    y = lax.bitcast_convert_type(
        jnp.full((16,), 0x5F3759DF, jnp.int32) - (iv >> 1), jnp.float32)
    for _ in range(3):
      y = y * (1.5 - 0.5 * vv * y * y)
    for r in range(_NV):
      x = rows_v[buf, t, pl.ds(16 * r, 16)]
      g = gb_v[0, pl.ds(16 * r, 16)]
      b = gb_v[1, pl.ds(16 * r, 16)]
      rows_v[buf, t, pl.ds(16 * r, 16)] = (x - mv) * y * g + b


def _fused(table, ids, ptt, gb):
  mesh = plsc.VectorSubcoreMesh(core_axis_name="c", subcore_axis_name="s")

  @functools.partial(
      pl.kernel,
      out_type=jax.ShapeDtypeStruct((_B, _S, _DIM), jnp.float32),
      mesh=mesh,
      scratch_types=[
          pltpu.VMEM((8, 128), jnp.int32),
          pltpu.VMEM((_SW, _DIM), jnp.float32),
          pltpu.VMEM((2, _DIM), jnp.float32),
          pltpu.VMEM((2, _CB * _SW, _DIM), jnp.float32),
          pltpu.SemaphoreType.DMA,
          pltpu.SemaphoreType.DMA,
      ],
  )
  def k(table_hbm, ids_hbm, ptt_hbm, gb_hbm, out_hbm,
        idx_v, ptt_v, gb_v, rows_v, g0, g1):
    wid = lax.axis_index("s") * _NC + lax.axis_index("c")
    s_base = wid * _SW
    pltpu.sync_copy(ids_hbm.at[wid], idx_v)
    pltpu.sync_copy(ptt_hbm.at[pl.ds(s_base, _SW)], ptt_v)
    pltpu.sync_copy(gb_hbm, gb_v)

    gsem = [g0, g1]

    def fire(c, buf):
      for j in range(_CB):
        bb = _CB * c + j
        pltpu.async_copy(
            table_hbm.at[idx_v.at[bb // 8, pl.ds((bb % 8) * _SW, _SW)]],
            rows_v.at[buf, pl.ds(_SW * j, _SW)],
            gsem[buf],
        )

    def drain(buf):
      for j in range(_CB):
        pltpu.make_async_copy(
            out_hbm.at[0, pl.ds(0, _SW)],
            rows_v.at[buf, pl.ds(_SW * j, _SW)],
            gsem[buf],
        ).wait()

    def write_out(c, buf):
      for j in range(_CB):
        pltpu.sync_copy(
            rows_v.at[buf, pl.ds(_SW * j, _SW)],
            out_hbm.at[_CB * c + j, pl.ds(s_base, _SW)],
        )

    fire(0, 0)

    def pair(p, carry):
      c0 = 2 * p
      c1 = c0 + 1
      fire(c1, 1)
      drain(0)
      _ln_chunk(rows_v, 0, ptt_v, gb_v)
      write_out(c0, 0)

      @pl.when(p < _NCHUNK // 2 - 1)
      def _():
        fire(c0 + 2, 0)

      drain(1)
      _ln_chunk(rows_v, 1, ptt_v, gb_v)
      write_out(c1, 1)
      return carry

    lax.fori_loop(0, _NCHUNK // 2, pair, 0)

  return k(table, ids, ptt, gb)


def kernel(input_ids, word_embeddings, position_embeddings,
           token_type_embeddings, ln_gamma, ln_beta):
  ids = (input_ids.astype(jnp.int32)
         .reshape(_B, _NW, _SW).transpose(1, 0, 2).reshape(_NW, 8, 128))
  ptt = position_embeddings + token_type_embeddings[0][None, :]
  gb = jnp.concatenate(
      [ln_gamma.reshape(1, _DIM), ln_beta.reshape(1, _DIM)], axis=0)
  return _fused(word_embeddings, ids, ptt, gb)


# speedup vs baseline: 2.8440x; 2.8440x over previous
import functools

import jax
import jax.numpy as jnp
from jax import lax
from jax.experimental import pallas as pl
from jax.experimental.pallas import tpu as pltpu
from jax.experimental.pallas import tpu_sc as plsc

_B = 64
_S = 512
_DIM = 768
_NTOK = _B * _S

_NC = 2
_NS = 16
_NW = _NC * _NS
_CHUNK = 64
_SIZES = (8, 16, 16, 16, 8)


def _sc_gather(table, ids, ntok):
  mesh = plsc.VectorSubcoreMesh(core_axis_name="c", subcore_axis_name="s")
  tok_per_w = ntok // _NW
  nchunk = tok_per_w // _CHUNK

  @functools.partial(
      pl.kernel,
      out_type=jax.ShapeDtypeStruct((ntok, _DIM), jnp.float32),
      mesh=mesh,
      scratch_types=[
          pltpu.VMEM((tok_per_w,), jnp.int32),
          pltpu.VMEM((2, _CHUNK, _DIM), jnp.float32),
          pltpu.SemaphoreType.DMA,
          pltpu.SemaphoreType.DMA,
          pltpu.SemaphoreType.DMA,
          pltpu.SemaphoreType.DMA,
      ],
  )
  def k(table_hbm, idx_hbm, out_hbm, idx_v, rows_v, g0, g1, o0, o1):
    wid = lax.axis_index("s") * _NC + lax.axis_index("c")
    base = wid * tok_per_w
    pltpu.sync_copy(idx_hbm.at[pl.ds(base, tok_per_w)], idx_v)

    gsem = [g0, g1]
    osem = [o0, o1]

    def gather(i):
      return pltpu.async_copy(
          table_hbm.at[idx_v.at[pl.ds(i * _CHUNK, _CHUNK)]],
          rows_v.at[i % 2],
          gsem[i % 2],
      )

    pend_g = [None, None]
    pend_o = [None, None]
    pend_g[0] = gather(0)
    for i in range(nchunk):
      b = i % 2
      nb = (i + 1) % 2
      if i + 1 < nchunk:
        if pend_o[nb] is not None:
          pend_o[nb].wait()
        pend_g[nb] = gather(i + 1)
      pend_g[b].wait()
      pend_o[b] = pltpu.async_copy(
          rows_v.at[b],
          out_hbm.at[pl.ds(base + i * _CHUNK, _CHUNK)],
          osem[b],
      )
    pend_o[0].wait()
    pend_o[1].wait()

  return k(table, ids)


def _tc_addln_chunk(words, ptt, gamma, beta, b0, nb, prev):
  def body(*refs):
    w_ref, p_ref, g_ref, b_ref = refs[:4]
    o_ref = refs[-1]
    x = w_ref[...] + p_ref[...]
    m = jnp.mean(x, axis=-1, keepdims=True)
    v = jnp.mean(x * x, axis=-1, keepdims=True) - m * m
    r = 1.0 / jnp.sqrt(v + 1e-12)
    o_ref[...] = (x - m) * (r * g_ref[...]) + b_ref[...]

  in_specs = [
      pl.BlockSpec((_S, _DIM), lambda i: (i, 0)),
      pl.BlockSpec((_S, _DIM), lambda i: (0, 0)),
      pl.BlockSpec((1, _DIM), lambda i: (0, 0)),
      pl.BlockSpec((1, _DIM), lambda i: (0, 0)),
  ]
  args = [words, ptt, gamma, beta]
  aliases = {}
  if prev is not None:
    in_specs.append(pl.BlockSpec(memory_space=pl.ANY))
    args.append(prev)
    aliases = {4: 0}

  return pl.pallas_call(
      body,
      grid=(nb,),
      in_specs=in_specs,
      out_specs=pl.BlockSpec((_S, _DIM), lambda i, b0=b0: (b0 + i, 0)),
      out_shape=jax.ShapeDtypeStruct((_NTOK, _DIM), jnp.float32),
      input_output_aliases=aliases,
  )(*args)


def kernel(input_ids, word_embeddings, position_embeddings,
           token_type_embeddings, ln_gamma, ln_beta):
  ids = input_ids.reshape(-1).astype(jnp.int32)
  ptt = position_embeddings + token_type_embeddings[0][None, :]
  gamma = ln_gamma.reshape(1, _DIM)
  beta = ln_beta.reshape(1, _DIM)

  offsets = [sum(_SIZES[:k]) for k in range(len(_SIZES))]
  words = [
      _sc_gather(word_embeddings,
                 lax.dynamic_slice_in_dim(ids, b0 * _S, nb * _S), nb * _S)
      for b0, nb in zip(offsets, _SIZES)
  ]
  out = None
  for w, b0, nb in zip(words, offsets, _SIZES):
    out = _tc_addln_chunk(w, ptt, gamma, beta, b0, nb, out)
  return out.reshape(_B, _S, _DIM)


# speedup vs baseline: 2.8792x; 1.0124x over previous
import functools

import jax
import jax.numpy as jnp
from jax import lax
from jax.experimental import pallas as pl
from jax.experimental.pallas import tpu as pltpu
from jax.experimental.pallas import tpu_sc as plsc

_B = 64
_S = 512
_DIM = 768
_NTOK = _B * _S

_NC = 2
_NS = 16
_NW = _NC * _NS
_CHUNK = 64
_SIZES = (16, 16, 16, 16)
_TB = 1024


def _sc_gather(table, ids, ntok):
  mesh = plsc.VectorSubcoreMesh(core_axis_name="c", subcore_axis_name="s")
  tok_per_w = ntok // _NW
  nchunk = tok_per_w // _CHUNK

  @functools.partial(
      pl.kernel,
      out_type=jax.ShapeDtypeStruct((ntok, _DIM), jnp.float32),
      mesh=mesh,
      scratch_types=[
          pltpu.VMEM((tok_per_w,), jnp.int32),
          pltpu.VMEM((2, _CHUNK, _DIM), jnp.float32),
          pltpu.SemaphoreType.DMA,
          pltpu.SemaphoreType.DMA,
          pltpu.SemaphoreType.DMA,
          pltpu.SemaphoreType.DMA,
      ],
  )
  def k(table_hbm, idx_hbm, out_hbm, idx_v, rows_v, g0, g1, o0, o1):
    wid = lax.axis_index("s") * _NC + lax.axis_index("c")
    base = wid * tok_per_w
    pltpu.sync_copy(idx_hbm.at[pl.ds(base, tok_per_w)], idx_v)

    gsem = [g0, g1]
    osem = [o0, o1]

    def gather(i):
      return pltpu.async_copy(
          table_hbm.at[idx_v.at[pl.ds(i * _CHUNK, _CHUNK)]],
          rows_v.at[i % 2],
          gsem[i % 2],
      )

    pend_g = [None, None]
    pend_o = [None, None]
    pend_g[0] = gather(0)
    for i in range(nchunk):
      b = i % 2
      nb = (i + 1) % 2
      if i + 1 < nchunk:
        if pend_o[nb] is not None:
          pend_o[nb].wait()
        pend_g[nb] = gather(i + 1)
      pend_g[b].wait()
      pend_o[b] = pltpu.async_copy(
          rows_v.at[b],
          out_hbm.at[pl.ds(base + i * _CHUNK, _CHUNK)],
          osem[b],
      )
    pend_o[0].wait()
    pend_o[1].wait()

  return k(table, ids)


def _tc_addln_chunk(words, ptt, gamma, beta, b0, nb, prev):
  def body(*refs):
    w_ref, p_ref, g_ref, b_ref = refs[:4]
    o_ref = refs[-1]
    x = w_ref[...] + p_ref[...]
    m = jnp.mean(x, axis=-1, keepdims=True)
    v = jnp.mean(x * x, axis=-1, keepdims=True) - m * m
    r = 1.0 / jnp.sqrt(v + 1e-12)
    o_ref[...] = (x - m) * (r * g_ref[...]) + b_ref[...]

  in_specs = [
      pl.BlockSpec((_TB, _DIM), lambda i: (i, 0)),
      pl.BlockSpec((_TB, _DIM), lambda i: (0, 0)),
      pl.BlockSpec((1, _DIM), lambda i: (0, 0)),
      pl.BlockSpec((1, _DIM), lambda i: (0, 0)),
  ]
  args = [words, ptt, gamma, beta]
  aliases = {}
  if prev is not None:
    in_specs.append(pl.BlockSpec(memory_space=pl.ANY))
    args.append(prev)
    aliases = {4: 0}

  return pl.pallas_call(
      body,
      grid=(nb * _S // _TB,),
      in_specs=in_specs,
      out_specs=pl.BlockSpec((_TB, _DIM), lambda i, b0=b0: (b0 * _S // _TB + i, 0)),
      out_shape=jax.ShapeDtypeStruct((_NTOK, _DIM), jnp.float32),
      input_output_aliases=aliases,
  )(*args)


def kernel(input_ids, word_embeddings, position_embeddings,
           token_type_embeddings, ln_gamma, ln_beta):
  ids = input_ids.reshape(-1).astype(jnp.int32)
  ptt1 = position_embeddings + token_type_embeddings[0][None, :]
  ptt = jnp.concatenate([ptt1] * (_TB // _S), axis=0)
  gamma = ln_gamma.reshape(1, _DIM)
  beta = ln_beta.reshape(1, _DIM)

  offsets = [sum(_SIZES[:k]) for k in range(len(_SIZES))]
  words = [
      _sc_gather(word_embeddings,
                 lax.dynamic_slice_in_dim(ids, b0 * _S, nb * _S), nb * _S)
      for b0, nb in zip(offsets, _SIZES)
  ]
  out = None
  for w, b0, nb in zip(words, offsets, _SIZES):
    out = _tc_addln_chunk(w, ptt, gamma, beta, b0, nb, out)
  return out.reshape(_B, _S, _DIM)


# speedup vs baseline: 2.8879x; 1.0030x over previous
import functools

import jax
import jax.numpy as jnp
from jax import lax
from jax.experimental import pallas as pl
from jax.experimental.pallas import tpu as pltpu
from jax.experimental.pallas import tpu_sc as plsc

_B = 64
_S = 512
_DIM = 768
_NTOK = _B * _S

_NC = 2
_NS = 16
_NW = _NC * _NS
_CHUNK = 64
_SIZES = (16, 16, 16, 16)
_TB = 2048


def _sc_gather(table, ids, ntok):
  mesh = plsc.VectorSubcoreMesh(core_axis_name="c", subcore_axis_name="s")
  tok_per_w = ntok // _NW
  nchunk = tok_per_w // _CHUNK

  @functools.partial(
      pl.kernel,
      out_type=jax.ShapeDtypeStruct((ntok, _DIM), jnp.float32),
      mesh=mesh,
      scratch_types=[
          pltpu.VMEM((tok_per_w,), jnp.int32),
          pltpu.VMEM((2, _CHUNK, _DIM), jnp.float32),
          pltpu.SemaphoreType.DMA,
          pltpu.SemaphoreType.DMA,
          pltpu.SemaphoreType.DMA,
          pltpu.SemaphoreType.DMA,
      ],
  )
  def k(table_hbm, idx_hbm, out_hbm, idx_v, rows_v, g0, g1, o0, o1):
    wid = lax.axis_index("s") * _NC + lax.axis_index("c")
    base = wid * tok_per_w
    pltpu.sync_copy(idx_hbm.at[pl.ds(base, tok_per_w)], idx_v)

    gsem = [g0, g1]
    osem = [o0, o1]

    def gather(i):
      return pltpu.async_copy(
          table_hbm.at[idx_v.at[pl.ds(i * _CHUNK, _CHUNK)]],
          rows_v.at[i % 2],
          gsem[i % 2],
      )

    pend_g = [None, None]
    pend_o = [None, None]
    pend_g[0] = gather(0)
    for i in range(nchunk):
      b = i % 2
      nb = (i + 1) % 2
      if i + 1 < nchunk:
        if pend_o[nb] is not None:
          pend_o[nb].wait()
        pend_g[nb] = gather(i + 1)
      pend_g[b].wait()
      pend_o[b] = pltpu.async_copy(
          rows_v.at[b],
          out_hbm.at[pl.ds(base + i * _CHUNK, _CHUNK)],
          osem[b],
      )
    pend_o[0].wait()
    pend_o[1].wait()

  return k(table, ids)


def _tc_addln_chunk(words, ptt, gamma, beta, b0, nb, prev):
  def body(*refs):
    w_ref, p_ref, g_ref, b_ref = refs[:4]
    o_ref = refs[-1]
    x = w_ref[...] + p_ref[...]
    m = jnp.mean(x, axis=-1, keepdims=True)
    v = jnp.mean(x * x, axis=-1, keepdims=True) - m * m
    r = 1.0 / jnp.sqrt(v + 1e-12)
    o_ref[...] = (x - m) * (r * g_ref[...]) + b_ref[...]

  in_specs = [
      pl.BlockSpec((_TB, _DIM), lambda i: (i, 0)),
      pl.BlockSpec((_TB, _DIM), lambda i: (0, 0)),
      pl.BlockSpec((1, _DIM), lambda i: (0, 0)),
      pl.BlockSpec((1, _DIM), lambda i: (0, 0)),
  ]
  args = [words, ptt, gamma, beta]
  aliases = {}
  if prev is not None:
    in_specs.append(pl.BlockSpec(memory_space=pl.ANY))
    args.append(prev)
    aliases = {4: 0}

  return pl.pallas_call(
      body,
      grid=(nb * _S // _TB,),
      in_specs=in_specs,
      out_specs=pl.BlockSpec((_TB, _DIM), lambda i, b0=b0: (b0 * _S // _TB + i, 0)),
      out_shape=jax.ShapeDtypeStruct((_NTOK, _DIM), jnp.float32),
      input_output_aliases=aliases,
  )(*args)


def kernel(input_ids, word_embeddings, position_embeddings,
           token_type_embeddings, ln_gamma, ln_beta):
  ids = input_ids.reshape(-1).astype(jnp.int32)
  ptt1 = position_embeddings + token_type_embeddings[0][None, :]
  ptt = jnp.concatenate([ptt1] * (_TB // _S), axis=0)
  gamma = ln_gamma.reshape(1, _DIM)
  beta = ln_beta.reshape(1, _DIM)

  offsets = [sum(_SIZES[:k]) for k in range(len(_SIZES))]
  words = [
      _sc_gather(word_embeddings,
                 lax.dynamic_slice_in_dim(ids, b0 * _S, nb * _S), nb * _S)
      for b0, nb in zip(offsets, _SIZES)
  ]
  out = None
  for w, b0, nb in zip(words, offsets, _SIZES):
    out = _tc_addln_chunk(w, ptt, gamma, beta, b0, nb, out)
  return out.reshape(_B, _S, _DIM)
